# baseline (speedup 1.0000x reference)
import functools

import jax
import jax.numpy as jnp
from jax import lax
from jax.experimental import pallas as pl
from jax.experimental.pallas import tpu as pltpu
from jax.experimental.pallas import tpu_sc as plsc

B = 16384
D = 64
NC = 2
NS = 16
NW = NC * NS
B_PER_W = B // NW
CHUNK = 128
N_CHUNKS = B_PER_W // CHUNK

BLK = 2048
GRID = B // BLK


def _sc_gather_body(uid_hbm, mid_hbm, e_uid_tab, e_mid_tab,
                    uid_out, mid_out,
                    uidx_v, midx_v, urows_v, mrows_v, usem, msem):
    wid = lax.axis_index("s") * NC + lax.axis_index("c")
    base = wid * B_PER_W
    cbase = wid * N_CHUNKS
    pltpu.sync_copy(uid_hbm.at[pl.ds(cbase, N_CHUNKS)], uidx_v)
    pltpu.sync_copy(mid_hbm.at[pl.ds(cbase, N_CHUNKS)], midx_v)
    copies = []
    for j in range(N_CHUNKS):
        copies.append(pltpu.async_copy(
            e_uid_tab.at[uidx_v.at[j]], urows_v.at[pl.ds(j * CHUNK, CHUNK)],
            usem))
        copies.append(pltpu.async_copy(
            e_mid_tab.at[midx_v.at[j]], mrows_v.at[pl.ds(j * CHUNK, CHUNK)],
            msem))
    for cp in copies:
        cp.wait()
    pltpu.sync_copy(urows_v, uid_out.at[pl.ds(base, B_PER_W)])
    pltpu.sync_copy(mrows_v, mid_out.at[pl.ds(base, B_PER_W)])


@functools.cache
def _sc_gather():
    return pl.kernel(
        _sc_gather_body,
        out_type=(jax.ShapeDtypeStruct((B, D), jnp.float32),
                  jax.ShapeDtypeStruct((B, D), jnp.float32)),
        mesh=plsc.VectorSubcoreMesh(core_axis_name="c", subcore_axis_name="s"),
        scratch_types=[
            pltpu.VMEM((N_CHUNKS, CHUNK), jnp.int32),
            pltpu.VMEM((N_CHUNKS, CHUNK), jnp.int32),
            pltpu.VMEM((B_PER_W, D), jnp.float32),
            pltpu.VMEM((B_PER_W, D), jnp.float32),
            pltpu.SemaphoreType.DMA,
            pltpu.SemaphoreType.DMA,
        ],
        compiler_params=pltpu.CompilerParams(use_tc_tiling_on_sc=False),
    )


def _tc_body(g_ref, a_ref, j_ref, eu_ref, em_ref,
             eg_tab, ea_tab, ej_tab, w_ref, b_ref, o_ref):
    em = em_ref[...]
    eu = eu_ref[...]
    w_top = w_ref[0:D, :]
    w_bot = w_ref[D:2 * D, :]
    m = jnp.dot(em, w_top, preferred_element_type=jnp.float32) + b_ref[...]
    acc_t = jnp.zeros((BLK, D), jnp.float32)
    acc_m = jnp.zeros((BLK, D), jnp.float32)
    for idx_ref, tab_ref, n in ((g_ref, eg_tab, 2),
                                (a_ref, ea_tab, 7),
                                (j_ref, ej_tab, 21)):
        tab = tab_ref[...]
        oh = (idx_ref[...] ==
              lax.broadcasted_iota(jnp.int32, (BLK, n), 1)).astype(jnp.float32)
        e_attr = jnp.dot(oh, tab, preferred_element_type=jnp.float32)
        tab_w = jnp.dot(tab, w_bot, preferred_element_type=jnp.float32)
        v = m + jnp.dot(oh, tab_w, preferred_element_type=jnp.float32)
        v = v - jnp.max(v, axis=1, keepdims=True)
        ev = jnp.exp(v)
        wgt = (ev / jnp.sum(ev, axis=1, keepdims=True)) * e_attr
        acc_t = acc_t + wgt
        acc_m = acc_m + wgt * wgt
    p = eu * acc_t + 0.5 * (acc_t * acc_t - acc_m)
    o_ref[...] = jnp.sum(p * em, axis=1, keepdims=True)


_tc_forward = pl.pallas_call(
    _tc_body,
    grid=(GRID,),
    in_specs=[
        pl.BlockSpec((BLK, 1), lambda i: (i, 0)),
        pl.BlockSpec((BLK, 1), lambda i: (i, 0)),
        pl.BlockSpec((BLK, 1), lambda i: (i, 0)),
        pl.BlockSpec((BLK, D), lambda i: (i, 0)),
        pl.BlockSpec((BLK, D), lambda i: (i, 0)),
        pl.BlockSpec((2, D), lambda i: (0, 0)),
        pl.BlockSpec((7, D), lambda i: (0, 0)),
        pl.BlockSpec((21, D), lambda i: (0, 0)),
        pl.BlockSpec((2 * D, D), lambda i: (0, 0)),
        pl.BlockSpec((1, D), lambda i: (0, 0)),
    ],
    out_specs=pl.BlockSpec((BLK, 1), lambda i: (i, 0)),
    out_shape=jax.ShapeDtypeStruct((B, 1), jnp.float32),
)


def kernel(uid, gender, age, job, mid, E_uid, E_gender, E_age, E_job, E_mid,
           att_W, att_b):
    uid2 = uid.reshape(B // CHUNK, CHUNK)
    mid2 = mid.reshape(B // CHUNK, CHUNK)
    e_uid, e_mid = _sc_gather()(uid2, mid2, E_uid, E_mid)
    return _tc_forward(gender.reshape(B, 1), age.reshape(B, 1),
                       job.reshape(B, 1), e_uid, e_mid,
                       E_gender, E_age, E_job, att_W, att_b.reshape(1, D))


# speedup vs baseline: 1.4836x; 1.4836x over previous
import functools

import jax
import jax.numpy as jnp
from jax import lax
from jax.experimental import pallas as pl
from jax.experimental.pallas import tpu as pltpu
from jax.experimental.pallas import tpu_sc as plsc

B = 16384
D = 64
NC = 2
NS = 16
NW = NC * NS
B_PER_W = B // NW
FIRE = 16
N_BATCH = B_PER_W // FIRE

BLK = 2048
GRID = B // BLK


def _sc_gather_body(uid_hbm, mid_hbm, uid_tab, mid_tab, uid_out, mid_out,
                    idx_v, rows_v, sem):
    wid = lax.axis_index("s") * NC + lax.axis_index("c")
    base = wid * B_PER_W

    def gather_table(ids_hbm, tab, out):
        pltpu.sync_copy(ids_hbm.at[wid], idx_v)

        def batch_body(c, _):
            vec = idx_v[pl.ds(c * FIRE, FIRE)]
            cps = []
            for k in range(FIRE):
                r = vec[k]
                cps.append(pltpu.async_copy(
                    tab.at[pl.ds(r, 1)], rows_v.at[pl.ds(c * FIRE + k, 1)],
                    sem))
            for cp in cps:
                cp.wait()
            return 0

        lax.fori_loop(0, N_BATCH, batch_body, 0)
        pltpu.sync_copy(rows_v, out.at[pl.ds(base, B_PER_W)])

    gather_table(uid_hbm, uid_tab, uid_out)
    gather_table(mid_hbm, mid_tab, mid_out)


@functools.cache
def _sc_gather():
    return pl.kernel(
        _sc_gather_body,
        out_type=(jax.ShapeDtypeStruct((B, D), jnp.float32),
                  jax.ShapeDtypeStruct((B, D), jnp.float32)),
        mesh=plsc.VectorSubcoreMesh(core_axis_name="c", subcore_axis_name="s"),
        scratch_types=[
            pltpu.VMEM((B_PER_W,), jnp.int32),
            pltpu.VMEM((B_PER_W, D), jnp.float32),
            pltpu.SemaphoreType.DMA,
        ],
        compiler_params=pltpu.CompilerParams(needs_layout_passes=False),
    )


def _tc_body(g_ref, a_ref, j_ref, eu_ref, em_ref,
             eg_tab, ea_tab, ej_tab, w_ref, b_ref, o_ref):
    em = em_ref[...]
    eu = eu_ref[...]
    w_top = w_ref[0:D, :]
    w_bot = w_ref[D:2 * D, :]
    m = jnp.dot(em, w_top, preferred_element_type=jnp.float32) + b_ref[...]
    acc_t = jnp.zeros((BLK, D), jnp.float32)
    acc_m = jnp.zeros((BLK, D), jnp.float32)
    for idx_ref, tab_ref, n in ((g_ref, eg_tab, 2),
                                (a_ref, ea_tab, 7),
                                (j_ref, ej_tab, 21)):
        tab = tab_ref[...]
        oh = (idx_ref[...] ==
              lax.broadcasted_iota(jnp.int32, (BLK, n), 1)).astype(jnp.float32)
        e_attr = jnp.dot(oh, tab, preferred_element_type=jnp.float32)
        tab_w = jnp.dot(tab, w_bot, preferred_element_type=jnp.float32)
        v = m + jnp.dot(oh, tab_w, preferred_element_type=jnp.float32)
        v = v - jnp.max(v, axis=1, keepdims=True)
        ev = jnp.exp(v)
        wgt = (ev / jnp.sum(ev, axis=1, keepdims=True)) * e_attr
        acc_t = acc_t + wgt
        acc_m = acc_m + wgt * wgt
    p = eu * acc_t + 0.5 * (acc_t * acc_t - acc_m)
    o_ref[...] = jnp.sum(p * em, axis=1, keepdims=True)


_tc_forward = pl.pallas_call(
    _tc_body,
    grid=(GRID,),
    in_specs=[
        pl.BlockSpec((BLK, 1), lambda i: (i, 0)),
        pl.BlockSpec((BLK, 1), lambda i: (i, 0)),
        pl.BlockSpec((BLK, 1), lambda i: (i, 0)),
        pl.BlockSpec((BLK, D), lambda i: (i, 0)),
        pl.BlockSpec((BLK, D), lambda i: (i, 0)),
        pl.BlockSpec((2, D), lambda i: (0, 0)),
        pl.BlockSpec((7, D), lambda i: (0, 0)),
        pl.BlockSpec((21, D), lambda i: (0, 0)),
        pl.BlockSpec((2 * D, D), lambda i: (0, 0)),
        pl.BlockSpec((1, D), lambda i: (0, 0)),
    ],
    out_specs=pl.BlockSpec((BLK, 1), lambda i: (i, 0)),
    out_shape=jax.ShapeDtypeStruct((B, 1), jnp.float32),
)


def kernel(uid, gender, age, job, mid, E_uid, E_gender, E_age, E_job, E_mid,
           att_W, att_b):
    uid2 = uid.reshape(NW, B_PER_W)
    mid2 = mid.reshape(NW, B_PER_W)
    e_uid, e_mid = _sc_gather()(uid2, mid2, E_uid, E_mid)
    return _tc_forward(gender.reshape(B, 1), age.reshape(B, 1),
                       job.reshape(B, 1), e_uid, e_mid,
                       E_gender, E_age, E_job, att_W, att_b.reshape(1, D))
